# baseline (speedup 1.0000x reference)
import functools

import jax
import jax.numpy as jnp
from jax import lax
from jax.experimental import pallas as pl
from jax.experimental.pallas import tpu as pltpu
from jax.experimental.pallas import tpu_sc as plsc

_NC, _NS, _L = 2, 16, 16
_NW = _NC * _NS
_CHUNK = 128
_D = 128

_f32 = jnp.float32
_i32 = jnp.int32


def _mesh():
    return plsc.VectorSubcoreMesh(
        core_axis_name="c", subcore_axis_name="s",
        num_cores=_NC, num_subcores=_NS)


def _deg_sc(col3, npad):
    steps = col3.shape[1]
    rpt = npad // _NS

    @functools.partial(
        pl.kernel,
        out_type=jax.ShapeDtypeStruct((_NC * npad,), _f32),
        mesh=_mesh(),
        scratch_types=[
            pltpu.VMEM((steps, _CHUNK), _i32),
            pltpu.VMEM((_CHUNK,), _f32),
            pltpu.VMEM((rpt,), _f32),
            pltpu.VMEM_SHARED((npad,), _f32),
        ],
    )
    def k(col_hbm, deg_hbm, cidx, ones_v, zbuf, deg_sh):
        cid = lax.axis_index("c")
        sid = lax.axis_index("s")
        wid = sid * _NC + cid

        def fill_ones(i, c):
            ones_v[pl.ds(i * _L, _L)] = jnp.ones((_L,), _f32)
            return c
        lax.fori_loop(0, _CHUNK // _L, fill_ones, 0)

        def fill_zeros(i, c):
            zbuf[pl.ds(i * _L, _L)] = jnp.zeros((_L,), _f32)
            return c
        lax.fori_loop(0, rpt // _L, fill_zeros, 0)
        pltpu.sync_copy(zbuf, deg_sh.at[pl.ds(sid * rpt, rpt)])
        pltpu.sync_copy(col_hbm.at[wid], cidx)
        plsc.subcore_barrier()

        def step(i, c):
            pltpu.sync_copy(ones_v, deg_sh.at[cidx.at[i]], add=True)
            return c
        lax.fori_loop(0, steps, step, 0)
        plsc.subcore_barrier()

        pltpu.sync_copy(deg_sh.at[pl.ds(sid * rpt, rpt)],
                        deg_hbm.at[pl.ds(cid * npad + sid * rpt, rpt)])

    return k(col3)


def _linear_tc(seq, W, dega, degb):
    n = seq.shape[0]
    blk = 1000

    def body(seq_ref, w_ref, da_ref, db_ref, y_ref, dis_ref):
        deg = da_ref[...] + db_ref[...] + 1.0
        dis = lax.rsqrt(deg)
        x = lax.dot_general(seq_ref[...], w_ref[...],
                            (((1,), (1,)), ((), ())),
                            preferred_element_type=_f32)
        y_ref[...] = x * dis
        dis_ref[...] = dis

    return pl.pallas_call(
        body,
        grid=(n // blk,),
        in_specs=[
            pl.BlockSpec((blk, _D), lambda i: (i, 0)),
            pl.BlockSpec((_D, _D), lambda i: (0, 0)),
            pl.BlockSpec((blk, 1), lambda i: (i, 0)),
            pl.BlockSpec((blk, 1), lambda i: (i, 0)),
        ],
        out_specs=[
            pl.BlockSpec((blk, _D), lambda i: (i, 0)),
            pl.BlockSpec((blk, 1), lambda i: (i, 0)),
        ],
        out_shape=[
            jax.ShapeDtypeStruct((n, _D), _f32),
            jax.ShapeDtypeStruct((n, 1), _f32),
        ],
    )(seq, W, dega, degb)


def _scatter_sc(y, row3, col3, npad):
    steps = row3.shape[1]
    gchunk = row3.shape[2]
    nbuf = 8
    seg = steps // 8
    nseg = steps // seg
    ngrp = seg // nbuf
    rpt = npad // _NS

    @functools.partial(
        pl.kernel,
        out_type=jax.ShapeDtypeStruct((_NC * npad, _D), _f32),
        mesh=_mesh(),
        scratch_types=[
            pltpu.VMEM((seg, gchunk), _i32),
            pltpu.VMEM((seg, gchunk), _i32),
        ] + [pltpu.VMEM((gchunk, _D), _f32)] * nbuf + [
            pltpu.VMEM_SHARED((npad, _D), _f32),
        ] + [pltpu.SemaphoreType.DMA] * nbuf,
    )
    def k(y_hbm, row_hbm, col_hbm, out_hbm, ridx, cidx, *rest):
        rows = rest[:nbuf]
        acc = rest[nbuf]
        sems = rest[nbuf + 1:]
        r0 = rows[0]
        cid = lax.axis_index("c")
        sid = lax.axis_index("s")
        wid = sid * _NC + cid

        def zrow(r, c):
            def zlane(j, c2):
                r0[r, pl.ds(j * _L, _L)] = jnp.zeros((_L,), _f32)
                return c2
            return lax.fori_loop(0, _D // _L, zlane, c)
        lax.fori_loop(0, gchunk, zrow, 0)

        def zacc(i, c):
            pltpu.sync_copy(r0, acc.at[pl.ds(sid * rpt + i * gchunk, gchunk)])
            return c
        lax.fori_loop(0, rpt // gchunk, zacc, 0)
        plsc.subcore_barrier()

        def seg_body(s, c):
            pltpu.sync_copy(row_hbm.at[wid, pl.ds(s * seg, seg)], ridx)
            pltpu.sync_copy(col_hbm.at[wid, pl.ds(s * seg, seg)], cidx)
            for b in range(nbuf):
                pltpu.async_copy(y_hbm.at[ridx.at[b]], rows[b], sems[b])

            def group(g, c2):
                base = g * nbuf
                for b in range(nbuf):
                    i = base + b
                    pltpu.make_async_copy(y_hbm.at[ridx.at[0]], rows[b],
                                          sems[b]).wait()
                    pltpu.sync_copy(rows[b], acc.at[cidx.at[i]], add=True)
                    nxt = jnp.minimum(i + nbuf, seg - 1)
                    pltpu.async_copy(y_hbm.at[ridx.at[nxt]], rows[b], sems[b])
                return c2
            lax.fori_loop(0, ngrp, group, 0)
            for b in range(nbuf):
                pltpu.make_async_copy(y_hbm.at[ridx.at[0]], rows[b],
                                      sems[b]).wait()
            return c
        lax.fori_loop(0, nseg, seg_body, 0)
        plsc.subcore_barrier()

        pltpu.sync_copy(acc.at[pl.ds(sid * rpt, rpt)],
                        out_hbm.at[pl.ds(cid * npad + sid * rpt, rpt)])

    return k(y, row3, col3)


def _finish_tc(acc0, acc1, y, dis, b2, pw2):
    n = y.shape[0]
    blk = 1000

    def body(a0_ref, a1_ref, y_ref, dis_ref, b_ref, pw_ref, out_ref):
        s = dis_ref[...] * (a0_ref[...] + a1_ref[...] + y_ref[...]) + b_ref[...]
        out_ref[...] = jnp.where(s >= 0, s, pw_ref[...] * s)

    return pl.pallas_call(
        body,
        grid=(n // blk,),
        in_specs=[
            pl.BlockSpec((blk, _D), lambda i: (i, 0)),
            pl.BlockSpec((blk, _D), lambda i: (i, 0)),
            pl.BlockSpec((blk, _D), lambda i: (i, 0)),
            pl.BlockSpec((blk, 1), lambda i: (i, 0)),
            pl.BlockSpec((1, _D), lambda i: (0, 0)),
            pl.BlockSpec((1, 1), lambda i: (0, 0)),
        ],
        out_specs=pl.BlockSpec((blk, _D), lambda i: (i, 0)),
        out_shape=jax.ShapeDtypeStruct((n, _D), _f32),
    )(acc0, acc1, y, dis, b2, pw2)


def kernel(seq, adj, W, b, prelu_w):
    n = seq.shape[0]
    row = adj[0].astype(_i32)
    col = adj[1].astype(_i32)
    e = row.shape[0]

    npad = ((n + _NS * _L - 1) // (_NS * _L)) * (_NS * _L)
    gchunk = 32
    egrp = _NW * gchunk * 64
    epad = (-e) % egrp
    row_p = jnp.concatenate([row, jnp.zeros((epad,), _i32)])
    col_p = jnp.concatenate([col, jnp.full((epad,), n, _i32)])
    dsteps = (e + epad) // (_NW * _CHUNK)
    gsteps = (e + epad) // (_NW * gchunk)

    deg2 = _deg_sc(col_p.reshape(_NW, dsteps, _CHUNK), npad)
    dega = deg2[:n, None]
    degb = deg2[npad:npad + n, None]

    y, dis = _linear_tc(seq, W, dega, degb)

    accs = _scatter_sc(y, row_p.reshape(_NW, gsteps, gchunk),
                       col_p.reshape(_NW, gsteps, gchunk), npad)

    return _finish_tc(accs[:n], accs[npad:npad + n], y, dis,
                      b.reshape(1, _D), prelu_w.reshape(1, 1))


# speedup vs baseline: 1.3415x; 1.3415x over previous
import functools

import jax
import jax.numpy as jnp
from jax import lax
from jax.experimental import pallas as pl
from jax.experimental.pallas import tpu as pltpu
from jax.experimental.pallas import tpu_sc as plsc

_NC, _NS, _L = 2, 16, 16
_NW = _NC * _NS
_CHUNK = 128
_D = 128

_f32 = jnp.float32
_i32 = jnp.int32


def _mesh():
    return plsc.VectorSubcoreMesh(
        core_axis_name="c", subcore_axis_name="s",
        num_cores=_NC, num_subcores=_NS)


def _deg_sc(col3, npad):
    steps = col3.shape[1]
    rpt = npad // _NS

    @functools.partial(
        pl.kernel,
        out_type=jax.ShapeDtypeStruct((_NC * npad,), _f32),
        mesh=_mesh(),
        scratch_types=[
            pltpu.VMEM((steps, _CHUNK), _i32),
            pltpu.VMEM((_CHUNK,), _f32),
            pltpu.VMEM((rpt,), _f32),
            pltpu.VMEM_SHARED((npad,), _f32),
        ],
    )
    def k(col_hbm, deg_hbm, cidx, ones_v, zbuf, deg_sh):
        cid = lax.axis_index("c")
        sid = lax.axis_index("s")
        wid = sid * _NC + cid

        def fill_ones(i, c):
            ones_v[pl.ds(i * _L, _L)] = jnp.ones((_L,), _f32)
            return c
        lax.fori_loop(0, _CHUNK // _L, fill_ones, 0)

        def fill_zeros(i, c):
            zbuf[pl.ds(i * _L, _L)] = jnp.zeros((_L,), _f32)
            return c
        lax.fori_loop(0, rpt // _L, fill_zeros, 0)
        pltpu.sync_copy(zbuf, deg_sh.at[pl.ds(sid * rpt, rpt)])
        pltpu.sync_copy(col_hbm.at[wid], cidx)
        plsc.subcore_barrier()

        def step(i, c):
            pltpu.sync_copy(ones_v, deg_sh.at[cidx.at[i]], add=True)
            return c
        lax.fori_loop(0, steps, step, 0)
        plsc.subcore_barrier()

        pltpu.sync_copy(deg_sh.at[pl.ds(sid * rpt, rpt)],
                        deg_hbm.at[pl.ds(cid * npad + sid * rpt, rpt)])

    return k(col3)


def _linear_tc(seq, W, dega, degb):
    n = seq.shape[0]
    blk = 1000

    def body(seq_ref, w_ref, da_ref, db_ref, y_ref, dis_ref):
        deg = da_ref[...] + db_ref[...] + 1.0
        dis = lax.rsqrt(deg)
        x = lax.dot_general(seq_ref[...], w_ref[...],
                            (((1,), (1,)), ((), ())),
                            preferred_element_type=_f32)
        y_ref[...] = x * dis
        dis_ref[...] = dis

    return pl.pallas_call(
        body,
        grid=(n // blk,),
        in_specs=[
            pl.BlockSpec((blk, _D), lambda i: (i, 0)),
            pl.BlockSpec((_D, _D), lambda i: (0, 0)),
            pl.BlockSpec((blk, 1), lambda i: (i, 0)),
            pl.BlockSpec((blk, 1), lambda i: (i, 0)),
        ],
        out_specs=[
            pl.BlockSpec((blk, _D), lambda i: (i, 0)),
            pl.BlockSpec((blk, 1), lambda i: (i, 0)),
        ],
        out_shape=[
            jax.ShapeDtypeStruct((n, _D), _f32),
            jax.ShapeDtypeStruct((n, 1), _f32),
        ],
    )(seq, W, dega, degb)


def _scatter_sc(y, row3, col3, npad):
    steps = row3.shape[1]
    gchunk = row3.shape[2]
    nbuf = 4
    seg = steps // 4
    nseg = steps // seg
    ngrp = seg // nbuf
    rpt = npad // _NS

    @functools.partial(
        pl.kernel,
        out_type=jax.ShapeDtypeStruct((_NC * npad, _D), _f32),
        mesh=_mesh(),
        scratch_types=[
            pltpu.VMEM((seg, gchunk), _i32),
            pltpu.VMEM((seg, gchunk), _i32),
        ] + [pltpu.VMEM((gchunk, _D), _f32)] * nbuf + [
            pltpu.VMEM_SHARED((npad, _D), _f32),
        ] + [pltpu.SemaphoreType.DMA] * nbuf,
    )
    def k(y_hbm, row_hbm, col_hbm, out_hbm, ridx, cidx, *rest):
        rows = rest[:nbuf]
        acc = rest[nbuf]
        sems = rest[nbuf + 1:]
        r0 = rows[0]
        cid = lax.axis_index("c")
        sid = lax.axis_index("s")
        wid = sid * _NC + cid

        def zrow(r, c):
            def zlane(j, c2):
                r0[r, pl.ds(j * _L, _L)] = jnp.zeros((_L,), _f32)
                return c2
            return lax.fori_loop(0, _D // _L, zlane, c)
        lax.fori_loop(0, gchunk, zrow, 0)

        def zacc(i, c):
            pltpu.sync_copy(r0, acc.at[pl.ds(sid * rpt + i * gchunk, gchunk)])
            return c
        lax.fori_loop(0, rpt // gchunk, zacc, 0)
        plsc.subcore_barrier()

        def seg_body(s, c):
            pltpu.sync_copy(row_hbm.at[wid, pl.ds(s * seg, seg)], ridx)
            pltpu.sync_copy(col_hbm.at[wid, pl.ds(s * seg, seg)], cidx)
            for b in range(nbuf):
                pltpu.async_copy(y_hbm.at[ridx.at[b]], rows[b], sems[b])
            def group(g, c2):
                base = g * nbuf
                for b in range(nbuf):
                    i = base + b
                    pltpu.make_async_copy(
                        y_hbm.at[ridx.at[i]], rows[b], sems[b]).wait()
                    pltpu.sync_copy(rows[b], acc.at[cidx.at[i]], add=True)
                    @pl.when(g < ngrp - 1)
                    def _():
                        pltpu.async_copy(
                            y_hbm.at[ridx.at[i + nbuf]], rows[b], sems[b])
                return c2
            lax.fori_loop(0, ngrp, group, 0)
            return c
        lax.fori_loop(0, nseg, seg_body, 0)
        plsc.subcore_barrier()

        pltpu.sync_copy(acc.at[pl.ds(sid * rpt, rpt)],
                        out_hbm.at[pl.ds(cid * npad + sid * rpt, rpt)])

    return k(y, row3, col3)


def _finish_tc(acc0, acc1, y, dis, b2, pw2):
    n = y.shape[0]
    blk = 1000

    def body(a0_ref, a1_ref, y_ref, dis_ref, b_ref, pw_ref, out_ref):
        s = dis_ref[...] * (a0_ref[...] + a1_ref[...] + y_ref[...]) + b_ref[...]
        out_ref[...] = jnp.where(s >= 0, s, pw_ref[...] * s)

    return pl.pallas_call(
        body,
        grid=(n // blk,),
        in_specs=[
            pl.BlockSpec((blk, _D), lambda i: (i, 0)),
            pl.BlockSpec((blk, _D), lambda i: (i, 0)),
            pl.BlockSpec((blk, _D), lambda i: (i, 0)),
            pl.BlockSpec((blk, 1), lambda i: (i, 0)),
            pl.BlockSpec((1, _D), lambda i: (0, 0)),
            pl.BlockSpec((1, 1), lambda i: (0, 0)),
        ],
        out_specs=pl.BlockSpec((blk, _D), lambda i: (i, 0)),
        out_shape=jax.ShapeDtypeStruct((n, _D), _f32),
    )(acc0, acc1, y, dis, b2, pw2)


def kernel(seq, adj, W, b, prelu_w):
    n = seq.shape[0]
    row = adj[0].astype(_i32)
    col = adj[1].astype(_i32)
    e = row.shape[0]

    npad = ((n + _NS * _L - 1) // (_NS * _L)) * (_NS * _L)
    gchunk = 64
    egrp = _NW * gchunk * 16
    epad = (-e) % egrp
    row_p = jnp.concatenate([row, jnp.zeros((epad,), _i32)])
    col_p = jnp.concatenate([col, jnp.full((epad,), n, _i32)])
    dsteps = (e + epad) // (_NW * _CHUNK)
    gsteps = (e + epad) // (_NW * gchunk)

    deg2 = _deg_sc(col_p.reshape(_NW, dsteps, _CHUNK), npad)
    dega = deg2[:n, None]
    degb = deg2[npad:npad + n, None]

    y, dis = _linear_tc(seq, W, dega, degb)

    accs = _scatter_sc(y, row_p.reshape(_NW, gsteps, gchunk),
                       col_p.reshape(_NW, gsteps, gchunk), npad)

    return _finish_tc(accs[:n], accs[npad:npad + n], y, dis,
                      b.reshape(1, _D), prelu_w.reshape(1, 1))
